# baseline (speedup 1.0000x reference)
import jax
import jax.numpy as jnp
from jax.experimental import pallas as pl
from jax.experimental.pallas import tpu as pltpu

_B = 8
_E = 256
_N = 1024
_HW = 1024
_COMMIT = 0.25
_LOSS_SCALE = (1.0 + _COMMIT) / float(_B * _E * _HW)


_NSUB = 4


def _vq_body(x_ref, cb_ref, q_ref, loss_ref):
    b = pl.program_id(0)
    cb = cb_ref[...]
    cbt_hi = cb.T.astype(jnp.bfloat16)
    for s in range(_NSUB):
        _vq_one(s, b, x_ref, cb, cbt_hi, q_ref, loss_ref)


def _vq_one(s, b, x_ref, cb, cbt_hi, q_ref, loss_ref):
    xb = x_ref[s]

    m = jnp.dot(cb, xb, preferred_element_type=jnp.float32)
    cnorm = jnp.sum(cb * cb, axis=1, keepdims=True)
    xnorm = jnp.sum(xb * xb, axis=0, keepdims=True)
    dist = (cnorm + xnorm) - 2.0 * m
    idx = jnp.argmin(dist, axis=0)

    eq16 = (jax.lax.broadcasted_iota(jnp.int16, (_N, _HW), 0)
            == idx.astype(jnp.int16)[None, :])
    onehot = jnp.where(eq16, jnp.bfloat16(1), jnp.bfloat16(0))
    q = jnp.dot(cbt_hi, onehot, preferred_element_type=jnp.float32)
    q_ref[s] = q

    part = jnp.sum((q - xb) ** 2)

    @pl.when(jnp.logical_and(b == 0, s == 0))
    def _init():
        loss_ref[0, 0] = 0.0

    loss_ref[0, 0] += part

    @pl.when(jnp.logical_and(b == _B // _NSUB - 1, s == _NSUB - 1))
    def _fin():
        loss_ref[0, 0] = loss_ref[0, 0] * _LOSS_SCALE


def kernel(x, codebook):
    xr = x.reshape(_B, _E, _HW)
    q, loss = pl.pallas_call(
        _vq_body,
        grid=(_B // _NSUB,),
        in_specs=[
            pl.BlockSpec((_NSUB, _E, _HW), lambda b: (b, 0, 0)),
            pl.BlockSpec((_N, _E), lambda b: (0, 0)),
        ],
        out_specs=[
            pl.BlockSpec((_NSUB, _E, _HW), lambda b: (b, 0, 0)),
            pl.BlockSpec(memory_space=pltpu.SMEM,
                         block_shape=(1, 1), index_map=lambda b: (0, 0)),
        ],
        out_shape=[
            jax.ShapeDtypeStruct((_B, _E, _HW), jnp.float32),
            jax.ShapeDtypeStruct((1, 1), jnp.float32),
        ],
    )(xr, codebook)
    return (q.reshape(_B, _E, 32, 32), loss[0, 0])
